# baseline (speedup 1.0000x reference)
import functools
import math

import jax
import jax.numpy as jnp
from jax import lax
from jax.experimental import pallas as pl
from jax.experimental.pallas import tpu as pltpu
from jax.experimental.pallas import tpu_sc as plsc

NUM_EMB = 1000000
DIM = 64
BATCH = 4096
SEQ = 200

NW = 32
NBLK = (NUM_EMB + 127) // 128
K1 = (NBLK + NW - 1) // NW
TRROWS = NUM_EMB // 2
BBLK = BATCH // 128


def _pos_embedding():
    position = jnp.arange(0, SEQ, dtype=jnp.float32)[:, None]
    div_term = jnp.arange(0, DIM, 2, dtype=jnp.float32)
    div_term = jnp.exp(div_term * (-math.log(10000.0) / DIM))
    pe = jnp.zeros((SEQ, DIM), dtype=jnp.float32)
    pe = pe.at[:, 0::2].set(jnp.sin(position * div_term))
    pe = pe.at[:, 1::2].set(jnp.cos(position * div_term))
    return pe


def kernel(data, table):
    info = plsc.get_sparse_core_info()
    nc, ns = info.num_cores, info.num_subcores
    assert nc * ns == NW

    tT = table.T
    idx4 = (data.astype(jnp.int32)
            .reshape(32, 128, 25, 8).transpose(2, 0, 3, 1))
    pe = _pos_embedding()

    mesh1 = plsc.VectorSubcoreMesh(core_axis_name="c", subcore_axis_name="s")

    @functools.partial(
        pl.kernel,
        mesh=mesh1,
        compiler_params=pltpu.CompilerParams(use_tc_tiling_on_sc=True,
                                             needs_layout_passes=False),
        out_type=jax.ShapeDtypeStruct((TRROWS, 128), jnp.float32),
        scratch_types=(
            [pltpu.VMEM((64, 128), jnp.float32)] * 4
            + [pltpu.VMEM((64, 128), jnp.float32)] * 4
            + [pltpu.SemaphoreType.DMA] * 8
        ),
    )
    def detile(tT_hbm, tr_hbm, *scr):
        sb = scr[0:4]
        ob = scr[4:8]
        gs = scr[8:12]
        os = scr[12:16]
        wid = lax.axis_index("s") * nc + lax.axis_index("c")
        dvecs = [jnp.arange(16, dtype=jnp.int32) + 16 * j for j in range(4)]

        def blk(k):
            return jnp.minimum(wid + k * NW, NBLK - 1)

        def start_in(k, sb, sem):
            pltpu.make_async_copy(
                tT_hbm.at[:, pl.ds(blk(k) * 128, 128)], sb, sem).start()

        def wait_in(k, sb, sem):
            pltpu.make_async_copy(
                tT_hbm.at[:, pl.ds(blk(k) * 128, 128)], sb, sem).wait()

        def start_out(k, ob, sem):
            b = blk(k)
            pltpu.make_async_copy(
                ob.at[pl.ds(0, 32)], tr_hbm.at[pl.ds(b * 64, 32)], sem).start()

            @pl.when(b < NBLK - 1)
            def _():
                pltpu.make_async_copy(
                    ob.at[pl.ds(32, 32)],
                    tr_hbm.at[pl.ds(b * 64 + 32, 32)], sem).start()

        def wait_out(k, ob, sem):
            b = blk(k)
            pltpu.make_async_copy(
                ob.at[pl.ds(0, 32)], tr_hbm.at[pl.ds(b * 64, 32)], sem).wait()

            @pl.when(b < NBLK - 1)
            def _():
                pltpu.make_async_copy(
                    ob.at[pl.ds(32, 32)],
                    tr_hbm.at[pl.ds(b * 64 + 32, 32)], sem).wait()

        def transpose(sb, ob):
            for t0 in range(0, 128, 4):
                vs = []
                for t in range(t0, t0 + 4):
                    ts = jnp.full((16,), t, jnp.int32)
                    for j in range(4):
                        vs.append(plsc.load_gather(sb, [dvecs[j], ts]))
                i = 0
                for t in range(t0, t0 + 4):
                    for j in range(4):
                        ob[t // 2, pl.ds((t % 2) * 64 + 16 * j, 16)] = vs[i]
                        i += 1

        for p in range(3):
            start_in(p, sb[p], gs[p])

        def unit(k, i):
            @pl.when(k + 3 < K1)
            def _():
                start_in(k + 3, sb[(i + 3) % 4], gs[(i + 3) % 4])

            wait_in(k, sb[i], gs[i])

            @pl.when(k >= 4)
            def _():
                wait_out(k - 4, ob[i], os[i])

            transpose(sb[i], ob[i])
            start_out(k, ob[i], os[i])

        def outer(g, carry):
            for i in range(4):
                unit(4 * g + i, i)
            return carry

        lax.fori_loop(0, K1 // 4, outer, 0)
        unit(K1 - 1, (K1 - 1) % 4)

        for k in range(K1 - 4, K1):
            wait_out(k, ob[k % 4], os[k % 4])

    tr = detile(tT)
    table_lin = tr.reshape(NUM_EMB, 64)

    mesh2 = plsc.VectorSubcoreMesh(core_axis_name="c", subcore_axis_name="s")

    @functools.partial(
        pl.kernel,
        mesh=mesh2,
        compiler_params=pltpu.CompilerParams(use_tc_tiling_on_sc=False,
                                             needs_layout_passes=False),
        out_type=jax.ShapeDtypeStruct((SEQ, 8, BBLK, 8, 128), jnp.float32),
        scratch_types=(
            [pltpu.VMEM((128,), jnp.int32)] * 4
            + [pltpu.VMEM((128, 64), jnp.float32)] * 4
            + [pltpu.VMEM((8, 8, 128), jnp.float32)] * 4
            + [pltpu.VMEM((SEQ, DIM), jnp.float32)]
            + [pltpu.SemaphoreType.DMA] * 8
        ),
    )
    def gather_add(idx_hbm, tab_hbm, pe_hbm, out_hbm, *scr):
        ix = scr[0:4]
        gb = scr[4:8]
        ob = scr[8:12]
        pe_v = scr[12]
        gs = scr[13:17]
        os = scr[17:21]
        w = lax.axis_index("s") * nc + lax.axis_index("c")
        pltpu.sync_copy(pe_hbm, pe_v)
        tvecs = [jnp.arange(16, dtype=jnp.int32) + 16 * g for g in range(8)]

        def load_idx(s, ix):
            pltpu.sync_copy(idx_hbm.at[s // 8, w, lax.rem(s, 8)], ix)

        def start_gather(ix, gb, sem):
            pltpu.make_async_copy(tab_hbm.at[ix], gb, sem).start()

        def wait_gather(ix, gb, sem):
            pltpu.make_async_copy(tab_hbm.at[ix], gb, sem).wait()

        def start_store(s, ob, sem):
            for dt in range(8):
                pltpu.make_async_copy(
                    ob.at[dt], out_hbm.at[s, dt, w], sem).start()

        def wait_store(s, ob, sem):
            for dt in range(8):
                pltpu.make_async_copy(
                    ob.at[dt], out_hbm.at[s, dt, w], sem).wait()

        def transpose_add(s, gb, ob):
            for j in range(4):
                pej = pe_v[s, pl.ds(16 * j, 16)]

                for dd in range(16):
                    d = 16 * j + dd
                    pvec = lax.gather(
                        pej, jnp.full((16, 1), dd, jnp.int32),
                        lax.GatherDimensionNumbers(
                            offset_dims=(), collapsed_slice_dims=(0,),
                            start_index_map=(0,)),
                        slice_sizes=(1,),
                        mode=lax.GatherScatterMode.PROMISE_IN_BOUNDS)
                    dsplat = jnp.full((16,), d, jnp.int32)
                    vs = [plsc.load_gather(gb, [tvecs[g], dsplat]) + pvec
                          for g in range(8)]
                    for g in range(8):
                        ob[d // 8, d % 8, pl.ds(16 * g, 16)] = vs[g]

        def unit(s, i):
            @pl.when(s + 3 < SEQ)
            def _():
                load_idx(s + 3, ix[(i + 3) % 4])
                start_gather(ix[(i + 3) % 4], gb[(i + 3) % 4], gs[(i + 3) % 4])

            wait_gather(ix[i], gb[i], gs[i])

            @pl.when(s >= 4)
            def _():
                wait_store(s - 4, ob[i], os[i])

            transpose_add(s, gb[i], ob[i])
            start_store(s, ob[i], os[i])

        for p in range(3):
            load_idx(p, ix[p])
            start_gather(ix[p], gb[p], gs[p])

        def outer(g, carry):
            for i in range(4):
                unit(4 * g + i, i)
            return carry

        lax.fori_loop(0, SEQ // 4, outer, 0)

        for s in range(SEQ - 4, SEQ):
            wait_store(s, ob[s % 4], os[s % 4])

    out5 = gather_add(idx4, table_lin, pe)
    return out5.transpose(2, 4, 0, 1, 3).reshape(BATCH, SEQ, DIM)


# speedup vs baseline: 1.6387x; 1.6387x over previous
import functools
import math

import jax
import jax.numpy as jnp
from jax import lax
from jax.experimental import pallas as pl
from jax.experimental.pallas import tpu as pltpu
from jax.experimental.pallas import tpu_sc as plsc

NUM_EMB = 1000000
DIM = 64
MAX_LEN = 5000
BATCH = 4096
SEQ = 200

ROWS = BATCH * SEQ
CHUNK = 128
NCHUNKS = ROWS // CHUNK
PE_EXT = SEQ + CHUNK


def _pos_embedding_ext():
    position = jnp.arange(0, SEQ, dtype=jnp.float32)[:, None]
    div_term = jnp.arange(0, DIM, 2, dtype=jnp.float32)
    div_term = jnp.exp(div_term * (-math.log(10000.0) / DIM))
    pe = jnp.zeros((SEQ, DIM), dtype=jnp.float32)
    pe = pe.at[:, 0::2].set(jnp.sin(position * div_term))
    pe = pe.at[:, 1::2].set(jnp.cos(position * div_term))
    return jnp.concatenate([pe, pe[:CHUNK]], axis=0)


def kernel(data, table):
    info = plsc.get_sparse_core_info()
    nc, ns = info.num_cores, info.num_subcores
    nw = nc * ns
    chunks_per_w = NCHUNKS // nw
    rows_per_w = chunks_per_w * CHUNK

    idx2d = data.reshape(NCHUNKS, CHUNK).astype(jnp.int32)
    pe_ext = _pos_embedding_ext()

    mesh = plsc.VectorSubcoreMesh(core_axis_name="c", subcore_axis_name="s")

    @functools.partial(
        pl.kernel,
        mesh=mesh,
        compiler_params=pltpu.CompilerParams(use_tc_tiling_on_sc=False),
        out_type=jax.ShapeDtypeStruct((ROWS, DIM), jnp.float32),
        scratch_types=[
            pltpu.VMEM((chunks_per_w, CHUNK), jnp.int32),
            pltpu.VMEM((PE_EXT, DIM), jnp.float32),
            pltpu.VMEM((CHUNK, DIM), jnp.float32),
            pltpu.VMEM((CHUNK, DIM), jnp.float32),
            pltpu.SemaphoreType.DMA,
            pltpu.SemaphoreType.DMA,
            pltpu.SemaphoreType.DMA,
            pltpu.SemaphoreType.DMA,
        ],
    )
    def emb_kernel(idx_hbm, table_hbm, pe_hbm, out_hbm,
                   idx_v, pe_v, buf0, buf1, gsem0, gsem1, ssem0, ssem1):
        wid = lax.axis_index("s") * nc + lax.axis_index("c")
        cbase = wid * chunks_per_w
        obase = wid * rows_per_w

        pltpu.sync_copy(idx_hbm.at[pl.ds(cbase, chunks_per_w)], idx_v)
        pltpu.sync_copy(pe_hbm, pe_v)

        def start_gather(c, buf, gsem):
            pltpu.make_async_copy(table_hbm.at[idx_v.at[c]], buf, gsem).start()

        def wait_gather(c, buf, gsem):
            pltpu.make_async_copy(table_hbm.at[idx_v.at[c]], buf, gsem).wait()

        def start_store(c, buf, ssem):
            pltpu.make_async_copy(
                buf, out_hbm.at[pl.ds(obase + c * CHUNK, CHUNK)], ssem).start()

        def wait_store(c, buf, ssem):
            pltpu.make_async_copy(
                buf, out_hbm.at[pl.ds(obase + c * CHUNK, CHUNK)], ssem).wait()

        def add_pe(buf, p0):
            def body(i, carry):
                s = p0 + i
                for j in range(4):
                    sl = pl.ds(j * 16, 16)
                    plsc.addupdate(buf.at[i, sl], pe_v[s, sl])
                return carry
            lax.fori_loop(0, CHUNK, body, 0, unroll=8)

        def step(c, buf, gsem, ssem, nbuf, ngsem, nssem):
            @pl.when(c + 1 < chunks_per_w)
            def _():
                @pl.when(c >= 1)
                def _():
                    wait_store(c - 1, nbuf, nssem)
                start_gather(c + 1, nbuf, ngsem)

            wait_gather(c, buf, gsem)
            add_pe(buf, lax.rem(c * CHUNK, SEQ))
            start_store(c, buf, ssem)

        start_gather(0, buf0, gsem0)

        def outer(g, carry):
            step(2 * g, buf0, gsem0, ssem0, buf1, gsem1, ssem1)
            step(2 * g + 1, buf1, gsem1, ssem1, buf0, gsem0, ssem0)
            return carry

        lax.fori_loop(0, chunks_per_w // 2, outer, 0)

        wait_store(chunks_per_w - 2, buf0, ssem0)
        wait_store(chunks_per_w - 1, buf1, ssem1)

    out = emb_kernel(idx2d, table, pe_ext)
    return out.reshape(BATCH, SEQ, DIM)
